# Initial kernel scaffold; baseline (speedup 1.0000x reference)
import jax
import jax.numpy as jnp
from jax.experimental import pallas as pl


def kernel(x, feature_embeddings, cls_token):
    raise NotImplementedError("write your pallas kernel here")


# speedup vs baseline: 33.4934x; 33.4934x over previous
import jax
import jax.numpy as jnp
from jax.experimental import pallas as pl

_BB = 256


def _bcast_body(comb_ref, out_ref):
    out_ref[...] = jnp.broadcast_to(comb_ref[...], out_ref.shape)


def kernel(x, feature_embeddings, cls_token):
    batch = x.shape[0]
    num_feats, d = feature_embeddings.shape
    seq = num_feats + 1
    comb = jnp.concatenate([cls_token[0], feature_embeddings], axis=0)
    comb_flat = comb.reshape(1, seq * d)
    out2d = pl.pallas_call(
        _bcast_body,
        grid=(batch // _BB,),
        in_specs=[pl.BlockSpec((1, seq * d), lambda i: (0, 0))],
        out_specs=pl.BlockSpec((_BB, seq * d), lambda i: (i, 0)),
        out_shape=jax.ShapeDtypeStruct((batch, seq * d), jnp.float32),
    )(comb_flat)
    return out2d.reshape(batch, seq, d)
